# baseline (speedup 1.0000x reference)
import jax
import jax.numpy as jnp
from jax.experimental import pallas as pl
from jax.experimental.pallas import tpu as pltpu

_N_CLASSES = 512
_TEMP = 1.0
_EPS = 1e-20
_B, _C, _H, _W = 64, 512, 32, 32
_HW = _H * _W
_NTOK = _B * _HW
_PER_BATCH = _C * _HW
_HALF_N = (_B * _PER_BATCH) // 2


def _rotl(v, d):
    return (v << jnp.uint32(d)) | (v >> jnp.uint32(32 - d))


def _threefry2x32(x0, x1):
    ks0 = jnp.uint32(0)
    ks1 = jnp.uint32(42)
    ks2 = jnp.uint32(0x1BD11BDA) ^ ks0 ^ ks1
    ks = (ks0, ks1, ks2)
    rot = ((13, 15, 26, 6), (17, 29, 16, 24))

    x0 = x0 + ks[0]
    x1 = x1 + ks[1]
    for j in range(5):
        for d in rot[j % 2]:
            x0 = x0 + x1
            x1 = _rotl(x1, d)
            x1 = x0 ^ x1
        x0 = x0 + ks[(j + 1) % 3]
        x1 = x1 + ks[(j + 2) % 3] + jnp.uint32(j + 1)
    return x0, x1


def _threefry_bits(counter):
    x0, x1 = _threefry2x32(jnp.zeros_like(counter), counter)
    return x0 ^ x1


def _bits_to_gumbel(bits):
    fb = (bits >> jnp.uint32(9)) | jnp.uint32(0x3F800000)
    u = jax.lax.bitcast_convert_type(fb, jnp.float32) - jnp.float32(1.0)
    return -jnp.log(-jnp.log(u + _EPS) + _EPS)


def _vq_kernel(x_ref, z_ref, ei_ref, hist_ref):
    b = pl.program_id(0)

    row = jax.lax.broadcasted_iota(jnp.int32, (_C, _HW), 0)
    col = jax.lax.broadcasted_iota(jnp.int32, (_C, _HW), 1)
    f = (b * _PER_BATCH + row + col * _C).astype(jnp.uint32)
    g = _bits_to_gumbel(_threefry_bits(f))

    cid = jax.lax.broadcasted_iota(jnp.int32, (_C, _HW), 0)
    t = (x_ref[0] + g) * (1.0 / _TEMP)
    m = jnp.max(t, axis=0, keepdims=True)
    e = jnp.exp(t - m)
    s = jnp.sum(e, axis=0, keepdims=True)
    z_ref[0] = e / s

    idx = jnp.min(jnp.where(t == m, cid, _N_CLASSES), axis=0, keepdims=True)
    ei_ref[0] = idx
    onehot = (cid == idx).astype(jnp.float32)
    hist_ref[0] = jnp.sum(onehot, axis=1, keepdims=True)


def _perp_kernel(hist_ref, perp_ref):
    p = jnp.sum(hist_ref[...], axis=0, keepdims=True) * (1.0 / _NTOK)
    perp = jnp.exp(-jnp.sum(p * jnp.log(p + 1e-10)))
    perp_ref[...] = jnp.broadcast_to(perp, (1, 1))


def kernel(x):
    x3 = x.reshape(_B, _C, _HW)
    z3, ei, hist = pl.pallas_call(
        _vq_kernel,
        grid=(_B,),
        in_specs=[
            pl.BlockSpec((1, _C, _HW), lambda b: (b, 0, 0)),
        ],
        out_specs=[
            pl.BlockSpec((1, _C, _HW), lambda b: (b, 0, 0)),
            pl.BlockSpec((1, 1, _HW), lambda b: (b, 0, 0)),
            pl.BlockSpec((1, _C, 1), lambda b: (b, 0, 0)),
        ],
        out_shape=[
            jax.ShapeDtypeStruct((_B, _C, _HW), jnp.float32),
            jax.ShapeDtypeStruct((_B, 1, _HW), jnp.int32),
            jax.ShapeDtypeStruct((_B, _C, 1), jnp.float32),
        ],
        compiler_params=pltpu.CompilerParams(
            dimension_semantics=("parallel",),
        ),
    )(x3)
    perp = pl.pallas_call(
        _perp_kernel,
        out_shape=jax.ShapeDtypeStruct((1, 1), jnp.float32),
    )(hist.reshape(_B, _C))
    z_q = z3.reshape(_B, _C, _H, _W)
    embed_ind = ei.reshape(_B, _H, _W)
    return (z_q, 0.0, embed_ind, perp[0, 0])


# speedup vs baseline: 1.4429x; 1.4429x over previous
import jax
import jax.numpy as jnp
from jax.experimental import pallas as pl
from jax.experimental.pallas import tpu as pltpu

_N_CLASSES = 512
_TEMP = 1.0
_EPS = 1e-20
_B, _C, _H, _W = 64, 512, 32, 32
_HW = _H * _W
_NTOK = _B * _HW
_PER_BATCH = _C * _HW
_HALF_N = (_B * _PER_BATCH) // 2


def _rotl(v, d):
    return (v << jnp.uint32(d)) | (v >> jnp.uint32(32 - d))


def _threefry2x32(x0, x1):
    ks0 = jnp.uint32(0)
    ks1 = jnp.uint32(42)
    ks2 = jnp.uint32(0x1BD11BDA) ^ ks0 ^ ks1
    ks = (ks0, ks1, ks2)
    rot = ((13, 15, 26, 6), (17, 29, 16, 24))

    x0 = x0 + ks[0]
    x1 = x1 + ks[1]
    for j in range(5):
        for d in rot[j % 2]:
            x0 = x0 + x1
            x1 = _rotl(x1, d)
            x1 = x0 ^ x1
        x0 = x0 + ks[(j + 1) % 3]
        x1 = x1 + ks[(j + 2) % 3] + jnp.uint32(j + 1)
    return x0, x1


def _threefry_bits(counter):
    x0, x1 = _threefry2x32(jnp.zeros_like(counter), counter)
    return x0 ^ x1


def _bits_to_gumbel(bits):
    fb = (bits >> jnp.uint32(9)) | jnp.uint32(0x3F800000)
    u = jax.lax.bitcast_convert_type(fb, jnp.float32) - jnp.float32(1.0)
    return -jnp.log(-jnp.log(u + _EPS) + _EPS)


_CHUNK = 16


def _vq_kernel(x_ref, z_ref, ei_ref, hist_ref):
    b = pl.program_id(0)
    base = b * _PER_BATCH

    rowc = jax.lax.broadcasted_iota(jnp.int32, (_CHUNK, _HW), 0)
    colc = jax.lax.broadcasted_iota(jnp.int32, (_CHUNK, _HW), 1)
    for r0 in range(0, _C, _CHUNK):
        f = (base + (rowc + r0) + colc * _C).astype(jnp.uint32)
        g = _bits_to_gumbel(_threefry_bits(f))
        z_ref[0, r0:r0 + _CHUNK, :] = x_ref[0, r0:r0 + _CHUNK, :] + g

    cid = jax.lax.broadcasted_iota(jnp.int32, (_C, _HW), 0)
    t = z_ref[0]
    m = jnp.max(t, axis=0, keepdims=True)
    e = jnp.exp(t - m)
    s = jnp.sum(e, axis=0, keepdims=True)

    idx = jnp.min(jnp.where(t == m, cid, _N_CLASSES), axis=0, keepdims=True)
    ei_ref[0] = idx
    onehot = (cid == idx).astype(jnp.float32)
    hist_ref[0] = jnp.sum(onehot, axis=1, keepdims=True)

    z_ref[0] = e / s


def _perp_kernel(hist_ref, perp_ref):
    p = jnp.sum(hist_ref[...], axis=0, keepdims=True) * (1.0 / _NTOK)
    perp = jnp.exp(-jnp.sum(p * jnp.log(p + 1e-10)))
    perp_ref[...] = jnp.broadcast_to(perp, (1, 1))


def kernel(x):
    x3 = x.reshape(_B, _C, _HW)
    z3, ei, hist = pl.pallas_call(
        _vq_kernel,
        grid=(_B,),
        in_specs=[
            pl.BlockSpec((1, _C, _HW), lambda b: (b, 0, 0)),
        ],
        out_specs=[
            pl.BlockSpec((1, _C, _HW), lambda b: (b, 0, 0)),
            pl.BlockSpec((1, 1, _HW), lambda b: (b, 0, 0)),
            pl.BlockSpec((1, _C, 1), lambda b: (b, 0, 0)),
        ],
        out_shape=[
            jax.ShapeDtypeStruct((_B, _C, _HW), jnp.float32),
            jax.ShapeDtypeStruct((_B, 1, _HW), jnp.int32),
            jax.ShapeDtypeStruct((_B, _C, 1), jnp.float32),
        ],
        compiler_params=pltpu.CompilerParams(
            dimension_semantics=("parallel",),
        ),
    )(x3)
    perp = pl.pallas_call(
        _perp_kernel,
        out_shape=jax.ShapeDtypeStruct((1, 1), jnp.float32),
    )(hist.reshape(_B, _C))
    z_q = z3.reshape(_B, _C, _H, _W)
    embed_ind = ei.reshape(_B, _H, _W)
    return (z_q, 0.0, embed_ind, perp[0, 0])


# speedup vs baseline: 1.4657x; 1.0158x over previous
import jax
import jax.numpy as jnp
from jax.experimental import pallas as pl
from jax.experimental.pallas import tpu as pltpu

_N_CLASSES = 512
_TEMP = 1.0
_EPS = 1e-20
_B, _C, _H, _W = 64, 512, 32, 32
_HW = _H * _W
_NTOK = _B * _HW
_PER_BATCH = _C * _HW
_HALF_N = (_B * _PER_BATCH) // 2


def _rotl(v, d):
    return (v << jnp.uint32(d)) | (v >> jnp.uint32(32 - d))


def _threefry2x32(x0, x1):
    ks0 = jnp.uint32(0)
    ks1 = jnp.uint32(42)
    ks2 = jnp.uint32(0x1BD11BDA) ^ ks0 ^ ks1
    ks = (ks0, ks1, ks2)
    rot = ((13, 15, 26, 6), (17, 29, 16, 24))

    x0 = x0 + ks[0]
    x1 = x1 + ks[1]
    for j in range(5):
        for d in rot[j % 2]:
            x0 = x0 + x1
            x1 = _rotl(x1, d)
            x1 = x0 ^ x1
        x0 = x0 + ks[(j + 1) % 3]
        x1 = x1 + ks[(j + 2) % 3] + jnp.uint32(j + 1)
    return x0, x1


def _threefry_bits(counter):
    x0, x1 = _threefry2x32(jnp.zeros_like(counter), counter)
    return x0 ^ x1


def _bits_to_gumbel(bits):
    fb = (bits >> jnp.uint32(9)) | jnp.uint32(0x3F800000)
    u = jax.lax.bitcast_convert_type(fb, jnp.float32) - jnp.float32(1.0)
    return -jnp.log(-jnp.log(u + _EPS) + _EPS)


_CHUNK = 16


def _vq_kernel(x_ref, z_ref, ei_ref, hist_ref, t_ref):
    b = pl.program_id(0)
    base = b * _PER_BATCH

    rowc = jax.lax.broadcasted_iota(jnp.int32, (_CHUNK, _HW), 0)
    colc = jax.lax.broadcasted_iota(jnp.int32, (_CHUNK, _HW), 1)
    colbase = colc * _C
    for r0 in range(0, _C, _CHUNK):
        f = ((base + r0) + rowc + colbase).astype(jnp.uint32)
        g = _bits_to_gumbel(_threefry_bits(f))
        t_ref[r0:r0 + _CHUNK, :] = x_ref[0, r0:r0 + _CHUNK, :] + g

    cid = jax.lax.broadcasted_iota(jnp.int32, (_C, _HW), 0)
    t = t_ref[...]
    m = jnp.max(t, axis=0, keepdims=True)
    e = jnp.exp(t - m)
    s = jnp.sum(e, axis=0, keepdims=True)
    z_ref[0] = e / s

    idx = jnp.min(jnp.where(t == m, cid, _N_CLASSES), axis=0, keepdims=True)
    ei_ref[0] = idx
    onehot = (cid == idx).astype(jnp.float32)
    hist_ref[0] = jnp.sum(onehot, axis=1, keepdims=True)


def _perp_kernel(hist_ref, perp_ref):
    p = jnp.sum(hist_ref[...], axis=0, keepdims=True) * (1.0 / _NTOK)
    perp = jnp.exp(-jnp.sum(p * jnp.log(p + 1e-10)))
    perp_ref[...] = jnp.broadcast_to(perp, (1, 1))


def kernel(x):
    x3 = x.reshape(_B, _C, _HW)
    z3, ei, hist = pl.pallas_call(
        _vq_kernel,
        grid=(_B,),
        in_specs=[
            pl.BlockSpec((1, _C, _HW), lambda b: (b, 0, 0)),
        ],
        out_specs=[
            pl.BlockSpec((1, _C, _HW), lambda b: (b, 0, 0)),
            pl.BlockSpec((1, 1, _HW), lambda b: (b, 0, 0)),
            pl.BlockSpec((1, _C, 1), lambda b: (b, 0, 0)),
        ],
        out_shape=[
            jax.ShapeDtypeStruct((_B, _C, _HW), jnp.float32),
            jax.ShapeDtypeStruct((_B, 1, _HW), jnp.int32),
            jax.ShapeDtypeStruct((_B, _C, 1), jnp.float32),
        ],
        compiler_params=pltpu.CompilerParams(
            dimension_semantics=("arbitrary",),
        ),
        scratch_shapes=[pltpu.VMEM((_C, _HW), jnp.float32)],
    )(x3)
    perp = pl.pallas_call(
        _perp_kernel,
        out_shape=jax.ShapeDtypeStruct((1, 1), jnp.float32),
    )(hist.reshape(_B, _C))
    z_q = z3.reshape(_B, _C, _H, _W)
    embed_ind = ei.reshape(_B, _H, _W)
    return (z_q, 0.0, embed_ind, perp[0, 0])


# speedup vs baseline: 1.4807x; 1.0103x over previous
import jax
import jax.numpy as jnp
from jax.experimental import pallas as pl
from jax.experimental.pallas import tpu as pltpu

_N_CLASSES = 512
_TEMP = 1.0
_EPS = 1e-20
_B, _C, _H, _W = 64, 512, 32, 32
_HW = _H * _W
_NTOK = _B * _HW
_PER_BATCH = _C * _HW
_HALF_N = (_B * _PER_BATCH) // 2


def _rotl(v, d):
    return (v << jnp.uint32(d)) | (v >> jnp.uint32(32 - d))


def _threefry2x32(x0, x1):
    ks0 = jnp.uint32(0)
    ks1 = jnp.uint32(42)
    ks2 = jnp.uint32(0x1BD11BDA) ^ ks0 ^ ks1
    ks = (ks0, ks1, ks2)
    rot = ((13, 15, 26, 6), (17, 29, 16, 24))

    x0 = x0 + ks[0]
    x1 = x1 + ks[1]
    for j in range(5):
        for d in rot[j % 2]:
            x0 = x0 + x1
            x1 = _rotl(x1, d)
            x1 = x0 ^ x1
        x0 = x0 + ks[(j + 1) % 3]
        x1 = x1 + ks[(j + 2) % 3] + jnp.uint32(j + 1)
    return x0, x1


def _threefry_bits(counter):
    x0, x1 = _threefry2x32(jnp.zeros_like(counter), counter)
    return x0 ^ x1


def _bits_to_gumbel(bits):
    fb = (bits >> jnp.uint32(9)) | jnp.uint32(0x3F800000)
    u = jax.lax.bitcast_convert_type(fb, jnp.float32) - jnp.float32(1.0)
    return -jnp.log(-jnp.log(u + _EPS) + _EPS)


_CHUNK = 16


def _vq_kernel(x_ref, z_ref, ei_ref, perp_ref, t_ref, hist_ref):
    b = pl.program_id(0)
    base = b * _PER_BATCH

    @pl.when(b == 0)
    def _init():
        hist_ref[...] = jnp.zeros_like(hist_ref)

    rowc = jax.lax.broadcasted_iota(jnp.int32, (_CHUNK, _HW), 0)
    colc = jax.lax.broadcasted_iota(jnp.int32, (_CHUNK, _HW), 1)
    colbase = colc * _C
    for r0 in range(0, _C, _CHUNK):
        f = ((base + r0) + rowc + colbase).astype(jnp.uint32)
        g = _bits_to_gumbel(_threefry_bits(f))
        t_ref[r0:r0 + _CHUNK, :] = x_ref[0, r0:r0 + _CHUNK, :] + g

    cid = jax.lax.broadcasted_iota(jnp.int32, (_C, _HW), 0)
    t = t_ref[...]
    m = jnp.max(t, axis=0, keepdims=True)
    e = jnp.exp(t - m)
    s = jnp.sum(e, axis=0, keepdims=True)
    z_ref[0] = e / s

    idx = jnp.min(jnp.where(t == m, cid, _N_CLASSES), axis=0, keepdims=True)
    ei_ref[0] = idx
    onehot = (cid == idx).astype(jnp.float32)
    hist_ref[...] += jnp.sum(onehot, axis=1, keepdims=True)

    @pl.when(b == _B - 1)
    def _finish():
        p = hist_ref[...] * (1.0 / _NTOK)
        perp = jnp.exp(-jnp.sum(p * jnp.log(p + 1e-10)))
        perp_ref[...] = jnp.broadcast_to(perp, (1, 1))


def kernel(x):
    x3 = x.reshape(_B, _C, _HW)
    z3, ei, perp = pl.pallas_call(
        _vq_kernel,
        grid=(_B,),
        in_specs=[
            pl.BlockSpec((1, _C, _HW), lambda b: (b, 0, 0)),
        ],
        out_specs=[
            pl.BlockSpec((1, _C, _HW), lambda b: (b, 0, 0)),
            pl.BlockSpec((1, 1, _HW), lambda b: (b, 0, 0)),
            pl.BlockSpec((1, 1), lambda b: (0, 0)),
        ],
        out_shape=[
            jax.ShapeDtypeStruct((_B, _C, _HW), jnp.float32),
            jax.ShapeDtypeStruct((_B, 1, _HW), jnp.int32),
            jax.ShapeDtypeStruct((1, 1), jnp.float32),
        ],
        compiler_params=pltpu.CompilerParams(
            dimension_semantics=("arbitrary",),
        ),
        scratch_shapes=[
            pltpu.VMEM((_C, _HW), jnp.float32),
            pltpu.VMEM((_C, 1), jnp.float32),
        ],
    )(x3)
    z_q = z3.reshape(_B, _C, _H, _W)
    embed_ind = ei.reshape(_B, _H, _W)
    return (z_q, 0.0, embed_ind, perp[0, 0])
